# Initial kernel scaffold; baseline (speedup 1.0000x reference)
import jax
import jax.numpy as jnp
from jax.experimental import pallas as pl


def kernel(x, edge_index, W1, b1, g1, be1, W2, b2, g2, be2, W3, b3):
    raise NotImplementedError("write your pallas kernel here")


# speedup vs baseline: 10.8482x; 10.8482x over previous
import functools

import jax
import jax.numpy as jnp
from jax import lax
from jax.experimental import pallas as pl
from jax.experimental.pallas import tpu as pltpu
from jax.experimental.pallas import tpu_sc as plsc

N = 10000
D = 128
H = 128
C = 2
E = 320000

NC = 2
NS = 16
NT = NC * NS
CH = 128
CPT = -(-E // (NT * CH))
EP = NT * CPT * CH
NP = 10240
RPT = NP // NS
RPB = RPT // CH

_F32 = jnp.float32


def _sc_mesh():
    return plsc.VectorSubcoreMesh(
        core_axis_name="c", subcore_axis_name="s",
        num_cores=NC, num_subcores=NS)


def _make_deg():
    @functools.partial(
        pl.kernel,
        out_type=[jax.ShapeDtypeStruct((NP,), _F32),
                  jax.ShapeDtypeStruct((NP,), _F32)],
        mesh=_sc_mesh(),
        scratch_types=[
            pltpu.VMEM((CPT, CH), jnp.int32),
            pltpu.VMEM((CH,), _F32),
            pltpu.VMEM((RPT,), _F32),
            pltpu.VMEM_SHARED((NP,), _F32),
        ],
    )
    def deg_kernel(dst_hbm, out0_hbm, out1_hbm, dst_v, ones_v, deg_v, acc):
        cid = lax.axis_index("c")
        sid = lax.axis_index("s")
        wid = cid * NS + sid
        r0 = sid * RPT

        for k in range(CH // 16):
            ones_v[pl.ds(k * 16, 16)] = jnp.ones((16,), _F32)

        def zb(i, carry):
            deg_v[pl.ds(i * 16, 16)] = jnp.zeros((16,), _F32)
            return carry

        lax.fori_loop(0, RPT // 16, zb, 0)
        pltpu.sync_copy(deg_v, acc.at[pl.ds(r0, RPT)])
        pltpu.sync_copy(dst_hbm.at[wid], dst_v)
        plsc.subcore_barrier()

        def body(j, carry):
            pltpu.sync_copy(ones_v, acc.at[dst_v.at[j]], add=True)
            return carry

        lax.fori_loop(0, CPT, body, 0)
        plsc.subcore_barrier()
        pltpu.sync_copy(acc.at[pl.ds(r0, RPT)], deg_v)

        @pl.when(cid == 0)
        def _():
            pltpu.sync_copy(deg_v, out0_hbm.at[pl.ds(r0, RPT)])

        @pl.when(cid != 0)
        def _():
            pltpu.sync_copy(deg_v, out1_hbm.at[pl.ds(r0, RPT)])

    return deg_kernel


def _make_agg(F):
    @functools.partial(
        pl.kernel,
        out_type=jax.ShapeDtypeStruct((NC, NP, F), _F32),
        mesh=_sc_mesh(),
        scratch_types=[
            pltpu.VMEM((CPT, CH), jnp.int32),
            pltpu.VMEM((CPT, CH), jnp.int32),
            pltpu.VMEM((CH, F), _F32),
            pltpu.VMEM_SHARED((NP, F), _F32),
            pltpu.SemaphoreType.DMA,
        ],
    )
    def agg_kernel(y_hbm, zeros_hbm, src_hbm, dst_hbm, out_hbm,
                   src_v, dst_v, buf, acc, sem):
        cid = lax.axis_index("c")
        sid = lax.axis_index("s")
        wid = cid * NS + sid
        r0 = sid * RPT

        @pl.when(cid == 0)
        def _():
            def ib(i, carry):
                pltpu.sync_copy(y_hbm.at[pl.ds(r0 + i * CH, CH)], buf)
                pltpu.sync_copy(buf, acc.at[pl.ds(r0 + i * CH, CH)])
                return carry

            lax.fori_loop(0, RPB, ib, 0)

        @pl.when(cid != 0)
        def _():
            pltpu.sync_copy(zeros_hbm, buf)

            def ib(i, carry):
                pltpu.sync_copy(buf, acc.at[pl.ds(r0 + i * CH, CH)])
                return carry

            lax.fori_loop(0, RPB, ib, 0)

        pltpu.sync_copy(src_hbm.at[wid], src_v)
        pltpu.sync_copy(dst_hbm.at[wid], dst_v)
        plsc.subcore_barrier()

        def body(j, carry):
            pltpu.async_copy(y_hbm.at[src_v.at[j]], buf, sem).wait()
            pltpu.sync_copy(buf, acc.at[dst_v.at[j]], add=True)
            return carry

        lax.fori_loop(0, CPT, body, 0)
        plsc.subcore_barrier()

        def ob(i, carry):
            pltpu.sync_copy(acc.at[pl.ds(r0 + i * CH, CH)], buf)
            pltpu.sync_copy(buf, out_hbm.at[cid, pl.ds(r0 + i * CH, CH)])
            return carry

        lax.fori_loop(0, RPB, ob, 0)

    return agg_kernel


@functools.cache
def _sc_calls():
    return _make_deg(), _make_agg(H)


def _tc_prep_body(x_ref, w_ref, d0_ref, d1_ref, y_ref, dinv_ref):
    deg = d0_ref[...] + d1_ref[...] + 1.0
    dinv = lax.rsqrt(deg)
    dinv_ref[...] = dinv
    hw = jnp.dot(x_ref[...], w_ref[...], preferred_element_type=_F32)
    y_ref[0:N, :] = hw * dinv[0:N, None]
    y_ref[N:NP, :] = jnp.zeros((NP - N, H), _F32)


def _tc_prep(x, w1, d0, d1):
    return pl.pallas_call(
        _tc_prep_body,
        out_shape=[
            jax.ShapeDtypeStruct((NP, H), _F32),
            jax.ShapeDtypeStruct((NP,), _F32),
        ],
    )(x, w1, d0, d1)


def _tc_mid_body(a0_ref, a1_ref, dinv_ref, b_ref, g_ref, be_ref, w_ref,
                 y_ref):
    dinv = dinv_ref[...]
    t = ((a0_ref[0:N, :] + a1_ref[0:N, :]) * dinv[0:N, None]) + b_ref[...]
    mu = jnp.mean(t, axis=0)
    var = jnp.mean((t - mu) ** 2, axis=0)
    hbn = (t - mu) * lax.rsqrt(var + 1e-5) * g_ref[...] + be_ref[...]
    hbn = jnp.maximum(hbn, 0.0)
    hw = jnp.dot(hbn, w_ref[...], preferred_element_type=_F32)
    y_ref[0:N, :] = hw * dinv[0:N, None]
    y_ref[N:NP, :] = jnp.zeros((NP - N, w_ref.shape[1]), _F32)


def _tc_mid(a0, a1, dinv, b, g, be, w):
    fout = w.shape[1]
    return pl.pallas_call(
        _tc_mid_body,
        out_shape=jax.ShapeDtypeStruct((NP, fout), _F32),
    )(a0, a1, dinv, b, g, be, w)


def _tc_last_body(a0_ref, a1_ref, dinv_ref, b_ref, g_ref, be_ref, y_ref):
    dinv = dinv_ref[...]
    t = ((a0_ref[0:N, :] + a1_ref[0:N, :]) * dinv[0:N, None]) + b_ref[...]
    mu = jnp.mean(t, axis=0)
    var = jnp.mean((t - mu) ** 2, axis=0)
    hbn = (t - mu) * lax.rsqrt(var + 1e-5) * g_ref[...] + be_ref[...]
    hbn = jnp.maximum(hbn, 0.0)
    y_ref[0:N, :] = hbn * dinv[0:N, None]
    y_ref[N:NP, :] = jnp.zeros((NP - N, H), _F32)


def _tc_last(a0, a1, dinv, b, g, be):
    return pl.pallas_call(
        _tc_last_body,
        out_shape=jax.ShapeDtypeStruct((NP, H), _F32),
    )(a0, a1, dinv, b, g, be)


def _tc_fin_body(a0_ref, a1_ref, dinv_ref, w_ref, b_ref, out_ref):
    dinv = dinv_ref[...]
    agg = (a0_ref[0:N, :] + a1_ref[0:N, :]) * dinv[0:N, None]
    out_ref[...] = (jnp.dot(agg, w_ref[...], preferred_element_type=_F32)
                    + b_ref[...])


def _tc_fin(a0, a1, dinv, w, b):
    return pl.pallas_call(
        _tc_fin_body,
        out_shape=jax.ShapeDtypeStruct((N, C), _F32),
    )(a0, a1, dinv, w, b)


def kernel(x, edge_index, W1, b1, g1, be1, W2, b2, g2, be2, W3, b3):
    src = edge_index[0]
    dst = edge_index[1]
    pad = EP - E
    fill = jnp.full((pad,), N, jnp.int32)
    src_p = jnp.concatenate([src, fill]).reshape(NT, CPT, CH)
    dst_p = jnp.concatenate([dst, fill]).reshape(NT, CPT, CH)
    z128 = jnp.zeros((CH, H), _F32)

    deg_call, agg128_call = _sc_calls()
    d0, d1 = deg_call(dst_p)
    y1, dinv = _tc_prep(x, W1, d0, d1)
    a = agg128_call(y1, z128, src_p, dst_p)
    y2 = _tc_mid(a[0], a[1], dinv, b1, g1, be1, W2)
    a = agg128_call(y2, z128, src_p, dst_p)
    y3 = _tc_last(a[0], a[1], dinv, b2, g2, be2)
    a = agg128_call(y3, z128, src_p, dst_p)
    return _tc_fin(a[0], a[1], dinv, W3, b3)
